# Initial kernel scaffold; baseline (speedup 1.0000x reference)
import jax
import jax.numpy as jnp
from jax.experimental import pallas as pl


def kernel(lig_pos, lig_feat, prot_pos, prot_feat, t, lig_batch, prot_batch, lig_W, lig_b, prot_W, prot_b, tm_W1, tm_b1, tm_W2, tm_b2, c1_rel_W, c1_rel_b, c1_root_W, c2_rel_W, c2_rel_b, c2_root_W, c3_rel_W, c3_rel_b, c3_root_W, ro_W1, ro_b1, ro_W2, ro_b2):
    raise NotImplementedError("write your pallas kernel here")


# speedup vs baseline: 1.5552x; 1.5552x over previous
import math

import jax
import jax.numpy as jnp
from jax.experimental import pallas as pl
from jax.experimental.pallas import tpu as pltpu

HID = 64
B = 64
NL = 2048
NP = 6144
N = NL + NP
R2 = 25.0
TI = 256
TJ = 256
NROW = N // TI


def _fused_body(bounds_ref,
                pos8_ref, posT8_ref, brow_ref, bcol_ref,
                ligf_ref, protf_ref, t_ref,
                ligW_ref, ligb_ref, protW_ref, protb_ref,
                tmW1_ref, tmb1_ref, tmW2_ref, tmb2_ref,
                relW_ref, relb_ref, rootW_ref,
                roW1_ref, rob1_ref, roW2_ref, rob2_ref,
                out_ref,
                h_ref, hr_ref):
    l = pl.program_id(0)

    @pl.when(l == 0)
    def _prologue():
        half = HID // 2
        e = math.log(10000.0) / (half - 1)
        freqs = jnp.exp(
            jax.lax.broadcasted_iota(jnp.int32, (1, half), 1
                                     ).astype(jnp.float32) * (-e))
        emb = t_ref[:, :] * freqs
        temb = jnp.concatenate([jnp.sin(emb), jnp.cos(emb)], axis=1)
        temb = jax.nn.silu(
            jnp.dot(temb, tmW1_ref[:, :], preferred_element_type=jnp.float32)
            + tmb1_ref[:, :])
        temb = (jnp.dot(temb, tmW2_ref[:, :], preferred_element_type=jnp.float32)
                + tmb2_ref[:, :])
        lb = brow_ref[0:NL, :]
        onehot = (lb == jax.lax.broadcasted_iota(jnp.int32, (1, B), 1)
                  ).astype(jnp.float32)
        t_node = jnp.dot(onehot, temb, preferred_element_type=jnp.float32)
        x_lig = (jnp.dot(ligf_ref[:, :], ligW_ref[:, :],
                         preferred_element_type=jnp.float32)
                 + ligb_ref[:, :] + t_node)
        x_prot = (jnp.dot(protf_ref[:, :], protW_ref[:, :],
                          preferred_element_type=jnp.float32)
                  + protb_ref[:, :])
        h_ref[0, 0:NL, :] = x_lig
        h_ref[0, NL:N, :] = x_prot

    cur = jax.lax.rem(l, 2)
    nxt = 1 - cur
    relb = relb_ref[pl.ds(l, 1), :]
    rootW = rootW_ref[l]
    hr_ref[:, :] = jnp.dot(h_ref[cur], relW_ref[l],
                           preferred_element_type=jnp.float32)

    def row_body(it, carry):
        r0 = it * TI
        h_tile = h_ref[cur, pl.ds(r0, TI), :]
        p_i = pos8_ref[pl.ds(r0, TI), :]
        p2_i = jnp.sum(p_i * p_i, axis=1, keepdims=True)
        b_i = brow_ref[pl.ds(r0, TI), :]
        row_ids = jax.lax.broadcasted_iota(jnp.int32, (TI, TJ), 0) + r0

        def col_body(jt, acc):
            j0 = jt * TJ
            pT_j = posT8_ref[:, pl.ds(j0, TJ)]
            p2_j = jnp.sum(pT_j * pT_j, axis=0, keepdims=True)
            cross = jnp.dot(p_i, pT_j, preferred_element_type=jnp.float32)
            d2 = p2_i + p2_j - 2.0 * cross
            b_j = bcol_ref[:, pl.ds(j0, TJ)]
            col_ids = jax.lax.broadcasted_iota(jnp.int32, (TI, TJ), 1) + j0
            mask = (d2 < R2) & (b_i == b_j) & (row_ids != col_ids)
            a = jnp.where(mask, 1.0, 0.0)
            hr_j = hr_ref[pl.ds(j0, TJ), :]
            return acc + jnp.dot(a, hr_j, preferred_element_type=jnp.float32)

        acc = jnp.zeros((TI, HID), jnp.float32)
        acc = jax.lax.fori_loop(bounds_ref[0, it], bounds_ref[1, it],
                                col_body, acc)
        acc = jax.lax.fori_loop(bounds_ref[2, it], bounds_ref[3, it],
                                col_body, acc)
        out_tile = (acc + relb
                    + jnp.dot(h_tile, rootW, preferred_element_type=jnp.float32))
        h_ref[nxt, pl.ds(r0, TI), :] = out_tile
        return carry

    jax.lax.fori_loop(0, NROW, row_body, 0)

    @pl.when(l == 2)
    def _epilogue():
        xl = h_ref[1, 0:NL, :]
        lb_col = bcol_ref[:, 0:NL]
        onehot = (jax.lax.broadcasted_iota(jnp.int32, (B, 1), 0) == lb_col
                  ).astype(jnp.float32)
        sums = jnp.dot(onehot, xl, preferred_element_type=jnp.float32)
        cnt = jnp.sum(onehot, axis=1, keepdims=True)
        mean = sums / jnp.maximum(cnt, 1.0)
        hmid = jax.nn.silu(
            jnp.dot(mean, roW1_ref[:, :], preferred_element_type=jnp.float32)
            + rob1_ref[:, :])
        out_ref[:, :] = (jnp.dot(hmid, roW2_ref[:, :],
                                 preferred_element_type=jnp.float32)
                         + rob2_ref[:, :])


@jax.jit
def kernel(lig_pos, lig_feat, prot_pos, prot_feat, t, lig_batch, prot_batch,
           lig_W, lig_b, prot_W, prot_b, tm_W1, tm_b1, tm_W2, tm_b2,
           c1_rel_W, c1_rel_b, c1_root_W, c2_rel_W, c2_rel_b, c2_root_W,
           c3_rel_W, c3_rel_b, c3_root_W, ro_W1, ro_b1, ro_W2, ro_b2):
    lig_batch = lig_batch.astype(jnp.int32)
    prot_batch = prot_batch.astype(jnp.int32)
    batch = jnp.concatenate([lig_batch, prot_batch])
    pos = jnp.concatenate([lig_pos, prot_pos], axis=0)
    pos8 = jnp.pad(pos, ((0, 0), (0, 5)))
    posT8 = pos8.T
    brow = batch[:, None]
    bcol = batch[None, :]

    bres = batch.reshape(NROW, TI)
    blo = bres[:, 0]
    bhi = bres[:, -1]
    lj0 = jnp.searchsorted(lig_batch, blo, side='left')
    lj1 = jnp.searchsorted(lig_batch, bhi, side='right')
    pj0 = jnp.searchsorted(prot_batch, blo, side='left')
    pj1 = jnp.searchsorted(prot_batch, bhi, side='right')
    ljt0 = lj0 // TJ
    ljt1 = jnp.where(lj1 > lj0, (lj1 + TJ - 1) // TJ, ljt0)
    nlt = NL // TJ
    pjt0 = nlt + pj0 // TJ
    pjt1 = jnp.where(pj1 > pj0, nlt + (pj1 + TJ - 1) // TJ, pjt0)
    bounds = jnp.stack([ljt0, ljt1, pjt0, pjt1]).astype(jnp.int32)

    relW = jnp.stack([c1_rel_W, c2_rel_W, c3_rel_W])
    relb = jnp.stack([c1_rel_b, c2_rel_b, c3_rel_b])
    rootW = jnp.stack([c1_root_W, c2_root_W, c3_root_W])

    smem = pl.BlockSpec(memory_space=pltpu.SMEM)
    out = pl.pallas_call(
        _fused_body,
        grid=(3,),
        in_specs=[smem] + [pl.BlockSpec(memory_space=pltpu.VMEM)] * 22,
        out_specs=pl.BlockSpec(memory_space=pltpu.VMEM),
        out_shape=jax.ShapeDtypeStruct((B, 1), jnp.float32),
        scratch_shapes=[
            pltpu.VMEM((2, N, HID), jnp.float32),
            pltpu.VMEM((N, HID), jnp.float32),
        ],
    )(bounds,
      pos8, posT8, brow, bcol,
      lig_feat, prot_feat, t[:, None],
      lig_W, lig_b[None, :], prot_W, prot_b[None, :],
      tm_W1, tm_b1[None, :], tm_W2, tm_b2[None, :],
      relW, relb, rootW,
      ro_W1, ro_b1[None, :], ro_W2, ro_b2[None, :])
    return out


# speedup vs baseline: 1.7032x; 1.0951x over previous
import math

import jax
import jax.numpy as jnp
from jax.experimental import pallas as pl
from jax.experimental.pallas import tpu as pltpu

HID = 64
B = 64
NL = 2048
NP = 6144
N = NL + NP
R2 = 25.0
TI = 256
TJ = 256
NROW = N // TI


def _fused_body(bounds_ref,
                pos8_ref, posT8_ref, brow_ref, bcol_ref,
                ligf_ref, protf_ref, t_ref,
                ligW_ref, ligb_ref, protW_ref, protb_ref,
                tmW1_ref, tmb1_ref, tmW2_ref, tmb2_ref,
                relW_ref, relb_ref, rootW_ref,
                roW1_ref, rob1_ref, roW2_ref, rob2_ref,
                out_ref,
                h_ref, hr_ref):
    l = pl.program_id(0)

    @pl.when(l == 0)
    def _prologue():
        half = HID // 2
        e = math.log(10000.0) / (half - 1)
        freqs = jnp.exp(
            jax.lax.broadcasted_iota(jnp.int32, (1, half), 1
                                     ).astype(jnp.float32) * (-e))
        emb = t_ref[:, :] * freqs
        temb = jnp.concatenate([jnp.sin(emb), jnp.cos(emb)], axis=1)
        temb = jax.nn.silu(
            jnp.dot(temb, tmW1_ref[:, :], preferred_element_type=jnp.float32)
            + tmb1_ref[:, :])
        temb = (jnp.dot(temb, tmW2_ref[:, :], preferred_element_type=jnp.float32)
                + tmb2_ref[:, :])
        lb = brow_ref[0:NL, :]
        onehot = (lb == jax.lax.broadcasted_iota(jnp.int32, (1, B), 1)
                  ).astype(jnp.float32)
        t_node = jnp.dot(onehot, temb, preferred_element_type=jnp.float32)
        x_lig = (jnp.dot(ligf_ref[:, :], ligW_ref[:, :],
                         preferred_element_type=jnp.float32)
                 + ligb_ref[:, :] + t_node)
        x_prot = (jnp.dot(protf_ref[:, :], protW_ref[:, :],
                          preferred_element_type=jnp.float32)
                  + protb_ref[:, :])
        h_ref[0, 0:NL, :] = x_lig
        h_ref[0, NL:N, :] = x_prot

    cur = jax.lax.rem(l, 2)
    nxt = 1 - cur
    relb = relb_ref[pl.ds(l, 1), :]
    rootW = rootW_ref[l]
    hr_ref[:, :] = jnp.dot(h_ref[cur], relW_ref[l],
                           preferred_element_type=jnp.float32)

    def row_body(it, carry):
        r0 = it * TI
        h_tile = h_ref[cur, pl.ds(r0, TI), :]
        p_i = pos8_ref[pl.ds(r0, TI), :]
        thr_i = R2 - jnp.sum(p_i * p_i, axis=1, keepdims=True)
        b_i = brow_ref[pl.ds(r0, TI), :]

        def col_step(jt):
            j0 = jt * TJ
            pT_j = posT8_ref[:, pl.ds(j0, TJ)]
            p2_j = jnp.sum(pT_j * pT_j, axis=0, keepdims=True)
            cross = jnp.dot(p_i, pT_j, preferred_element_type=jnp.float32)
            b_j = bcol_ref[:, pl.ds(j0, TJ)]
            mask = (p2_j - 2.0 * cross < thr_i) & (b_i == b_j)
            a = jnp.where(mask, 1.0, 0.0)
            hr_j = hr_ref[pl.ds(j0, TJ), :]
            return jnp.dot(a, hr_j, preferred_element_type=jnp.float32)

        def range_sum(lo, hi, accs):
            half = (hi - lo) // 2

            def body2(k, accs):
                a0, a1 = accs
                jt = lo + 2 * k
                return a0 + col_step(jt), a1 + col_step(jt + 1)

            a0, a1 = jax.lax.fori_loop(0, half, body2, accs)
            a0 = jax.lax.cond(lo + 2 * half < hi,
                              lambda a: a + col_step(hi - 1),
                              lambda a: a, a0)
            return a0, a1

        accs = (-hr_ref[pl.ds(r0, TI), :], jnp.zeros((TI, HID), jnp.float32))
        accs = range_sum(bounds_ref[0, it], bounds_ref[1, it], accs)
        accs = range_sum(bounds_ref[2, it], bounds_ref[3, it], accs)
        out_tile = (accs[0] + accs[1] + relb
                    + jnp.dot(h_tile, rootW, preferred_element_type=jnp.float32))
        h_ref[nxt, pl.ds(r0, TI), :] = out_tile
        return carry

    jax.lax.fori_loop(0, NROW, row_body, 0)

    @pl.when(l == 2)
    def _epilogue():
        xl = h_ref[1, 0:NL, :]
        lb_col = bcol_ref[:, 0:NL]
        onehot = (jax.lax.broadcasted_iota(jnp.int32, (B, 1), 0) == lb_col
                  ).astype(jnp.float32)
        sums = jnp.dot(onehot, xl, preferred_element_type=jnp.float32)
        cnt = jnp.sum(onehot, axis=1, keepdims=True)
        mean = sums / jnp.maximum(cnt, 1.0)
        hmid = jax.nn.silu(
            jnp.dot(mean, roW1_ref[:, :], preferred_element_type=jnp.float32)
            + rob1_ref[:, :])
        out_ref[:, :] = (jnp.dot(hmid, roW2_ref[:, :],
                                 preferred_element_type=jnp.float32)
                         + rob2_ref[:, :])


@jax.jit
def kernel(lig_pos, lig_feat, prot_pos, prot_feat, t, lig_batch, prot_batch,
           lig_W, lig_b, prot_W, prot_b, tm_W1, tm_b1, tm_W2, tm_b2,
           c1_rel_W, c1_rel_b, c1_root_W, c2_rel_W, c2_rel_b, c2_root_W,
           c3_rel_W, c3_rel_b, c3_root_W, ro_W1, ro_b1, ro_W2, ro_b2):
    lig_batch = lig_batch.astype(jnp.int32)
    prot_batch = prot_batch.astype(jnp.int32)
    batch = jnp.concatenate([lig_batch, prot_batch])
    pos = jnp.concatenate([lig_pos, prot_pos], axis=0)
    pos8 = jnp.pad(pos, ((0, 0), (0, 5)))
    posT8 = pos8.T
    brow = batch[:, None]
    bcol = batch[None, :]

    bres = batch.reshape(NROW, TI)
    blo = bres[:, 0]
    bhi = bres[:, -1]
    lj0 = jnp.searchsorted(lig_batch, blo, side='left')
    lj1 = jnp.searchsorted(lig_batch, bhi, side='right')
    pj0 = jnp.searchsorted(prot_batch, blo, side='left')
    pj1 = jnp.searchsorted(prot_batch, bhi, side='right')
    ljt0 = lj0 // TJ
    ljt1 = jnp.where(lj1 > lj0, (lj1 + TJ - 1) // TJ, ljt0)
    nlt = NL // TJ
    pjt0 = nlt + pj0 // TJ
    pjt1 = jnp.where(pj1 > pj0, nlt + (pj1 + TJ - 1) // TJ, pjt0)
    bounds = jnp.stack([ljt0, ljt1, pjt0, pjt1]).astype(jnp.int32)

    relW = jnp.stack([c1_rel_W, c2_rel_W, c3_rel_W])
    relb = jnp.stack([c1_rel_b, c2_rel_b, c3_rel_b])
    rootW = jnp.stack([c1_root_W, c2_root_W, c3_root_W])

    smem = pl.BlockSpec(memory_space=pltpu.SMEM)
    out = pl.pallas_call(
        _fused_body,
        grid=(3,),
        in_specs=[smem] + [pl.BlockSpec(memory_space=pltpu.VMEM)] * 22,
        out_specs=pl.BlockSpec(memory_space=pltpu.VMEM),
        out_shape=jax.ShapeDtypeStruct((B, 1), jnp.float32),
        scratch_shapes=[
            pltpu.VMEM((2, N, HID), jnp.float32),
            pltpu.VMEM((N, HID), jnp.float32),
        ],
    )(bounds,
      pos8, posT8, brow, bcol,
      lig_feat, prot_feat, t[:, None],
      lig_W, lig_b[None, :], prot_W, prot_b[None, :],
      tm_W1, tm_b1[None, :], tm_W2, tm_b2[None, :],
      relW, relb, rootW,
      ro_W1, ro_b1[None, :], ro_W2, ro_b2[None, :])
    return out
